# baseline (speedup 1.0000x reference)
import jax
import jax.numpy as jnp
from jax.experimental import pallas as pl

_MEM_SLOTS = 262144


def _bbpm_kernel(x_ref, w_ref, b_ref, out_ref, slots_ref):
    B, T, D = x_ref.shape
    x = x_ref[...]
    s = jnp.sum(x * 1000.0, axis=-1)
    h = jnp.floor(s).astype(jnp.int32)
    slots = jnp.mod(h, _MEM_SLOTS)
    slots_f = slots.astype(jnp.float32)
    slots_ref[...] = slots_f
    qcol = slots_f[:, T - 1][:, None]

    t_idx = jax.lax.broadcasted_iota(jnp.int32, (B, T), 1)
    valid = t_idx < (T - 1)

    def body(bp, acc):
        srow = slots_ref[pl.ds(bp, 1), :]
        mask = jnp.where((qcol == srow) & valid, 1.0, 0.0)
        xb = x_ref[pl.ds(bp, 1)].reshape(T, D)
        hi = xb.astype(jnp.bfloat16)
        lo = (xb - hi.astype(jnp.float32)).astype(jnp.bfloat16)
        mb = mask.astype(jnp.bfloat16)
        dot = lambda a, bb: jax.lax.dot_general(
            a, bb, (((1,), (0,)), ((), ())),
            preferred_element_type=jnp.float32)
        return acc + dot(mb, hi) + dot(mb, lo)

    retrieved = jax.lax.fori_loop(
        0, B, body, jnp.zeros((B, D), jnp.float32))

    out = jax.lax.dot_general(
        retrieved, w_ref[...], (((1,), (1,)), ((), ())),
        precision=jax.lax.Precision.HIGHEST,
        preferred_element_type=jnp.float32)
    out_ref[...] = out + b_ref[...][None, :]


def kernel(x, hx_list, W, b):
    del hx_list
    B, T, D = x.shape
    from jax.experimental.pallas import tpu as pltpu
    return pl.pallas_call(
        _bbpm_kernel,
        out_shape=jax.ShapeDtypeStruct((B, D), x.dtype),
        scratch_shapes=[pltpu.VMEM((B, T), jnp.float32)],
    )(x, W, b)


# speedup vs baseline: 1.0382x; 1.0382x over previous
import jax
import jax.numpy as jnp
from jax.experimental import pallas as pl

_MEM_SLOTS = 262144


def _bbpm_kernel(x_ref, w_ref, b_ref, out_ref):
    B, T, D = x_ref.shape
    x = x_ref[...]
    s = jnp.sum(x * 1000.0, axis=-1)
    h = jnp.floor(s).astype(jnp.int32)
    slots = jnp.mod(h, _MEM_SLOTS)
    slots_f = slots.astype(jnp.float32)
    qcol = slots_f[:, T - 1][:, None]

    t_idx = jax.lax.broadcasted_iota(jnp.int32, (B, T), 1)
    valid = t_idx < (T - 1)

    dot = lambda a, bb: jax.lax.dot_general(
        a, bb, (((1,), (0,)), ((), ())),
        preferred_element_type=jnp.float32)

    hi = x.astype(jnp.bfloat16)
    lo = (x - hi.astype(jnp.float32)).astype(jnp.bfloat16)

    acc = jnp.zeros((B, D), jnp.float32)
    for bp in range(B):
        srow = slots_f[bp][None, :]
        mask = jnp.where((qcol == srow) & valid, 1.0, 0.0)
        mb = mask.astype(jnp.bfloat16)
        acc = acc + dot(mb, hi[bp]) + dot(mb, lo[bp])

    out = jax.lax.dot_general(
        acc, w_ref[...], (((1,), (1,)), ((), ())),
        precision=jax.lax.Precision.HIGHEST,
        preferred_element_type=jnp.float32)
    out_ref[...] = out + b_ref[...][None, :]


def kernel(x, hx_list, W, b):
    del hx_list
    B, T, D = x.shape
    return pl.pallas_call(
        _bbpm_kernel,
        out_shape=jax.ShapeDtypeStruct((B, D), x.dtype),
    )(x, W, b)


# speedup vs baseline: 16.4360x; 15.8313x over previous
import jax
import jax.numpy as jnp
from jax.experimental import pallas as pl

_MEM_SLOTS = 262144


def _bbpm_kernel(x_ref, w_ref, b_ref, out_ref):
    B, T, D = x_ref.shape
    x = x_ref[...]
    writes = x.reshape(B * T, D)
    s = jnp.sum(writes * 1000.0, axis=-1)
    h = jnp.floor(s).astype(jnp.int32)
    slots = jnp.mod(h, _MEM_SLOTS)
    srow = slots.astype(jnp.float32)[None, :]

    q = x[:, T - 1, :]
    sq = jnp.sum(q * 1000.0, axis=-1)
    hq = jnp.floor(sq).astype(jnp.int32)
    qcol = jnp.mod(hq, _MEM_SLOTS).astype(jnp.float32)[:, None]

    j = jax.lax.broadcasted_iota(jnp.int32, (1, B * T), 1)
    valid = jnp.mod(j, T) != (T - 1)
    mask = jnp.where((qcol == srow) & valid, 1.0, 0.0)

    hi = writes.astype(jnp.bfloat16)
    lo = (writes - hi.astype(jnp.float32)).astype(jnp.bfloat16)
    mb = mask.astype(jnp.bfloat16)
    dot = lambda a, bb: jax.lax.dot_general(
        a, bb, (((1,), (0,)), ((), ())), preferred_element_type=jnp.float32)
    retrieved = dot(mb, hi) + dot(mb, lo)

    out = jax.lax.dot_general(
        retrieved, w_ref[...], (((1,), (1,)), ((), ())),
        precision=jax.lax.Precision.HIGHEST,
        preferred_element_type=jnp.float32)
    out_ref[...] = out + b_ref[...][None, :]


def kernel(x, hx_list, W, b):
    del hx_list
    B, T, D = x.shape
    return pl.pallas_call(
        _bbpm_kernel,
        out_shape=jax.ShapeDtypeStruct((B, D), x.dtype),
    )(x, W, b)
